# Initial kernel scaffold; baseline (speedup 1.0000x reference)
import jax
import jax.numpy as jnp
from jax.experimental import pallas as pl


def kernel(seq, token_table, pos_table):
    raise NotImplementedError("write your pallas kernel here")


# speedup vs baseline: 1.4244x; 1.4244x over previous
import functools

import jax
import jax.numpy as jnp
from jax import lax
from jax.experimental import pallas as pl
from jax.experimental.pallas import tpu as pltpu
from jax.experimental.pallas import tpu_sc as plsc

NC = 2
NS = 16
NW = NC * NS

BATCH = 4096
MAX_LEN = 200
DEPTH = 32
NROWS = BATCH * MAX_LEN
ROWS_PER_W = NROWS // NW

GROUP = 100
CHUNK = 1600
GPC = CHUNK // GROUP
NCHUNK = ROWS_PER_W // CHUNK
REPS = CHUNK // MAX_LEN

_mesh = plsc.VectorSubcoreMesh(core_axis_name="c", subcore_axis_name="s")


@functools.partial(
    pl.kernel,
    out_type=jax.ShapeDtypeStruct((NROWS, DEPTH), jnp.float32),
    mesh=_mesh,
    compiler_params=pltpu.CompilerParams(use_tc_tiling_on_sc=False),
    scratch_types=[
        pltpu.VMEM((GPC, GROUP), jnp.int32),
        pltpu.VMEM((CHUNK, DEPTH), jnp.float32),
        pltpu.VMEM((MAX_LEN, DEPTH), jnp.float32),
        pltpu.SemaphoreType.DMA,
    ],
)
def _embed(seq_hbm, tok_hbm, pos_hbm, out_hbm, idx_v, rows_v, pos_v, sem):
    wid = lax.axis_index("s") * NC + lax.axis_index("c")
    base = wid * ROWS_PER_W

    pltpu.sync_copy(pos_hbm, pos_v)

    def chunk_body(ci, carry):
        off = pl.multiple_of(base + ci * CHUNK, CHUNK)
        pltpu.sync_copy(seq_hbm.at[pl.ds(pl.multiple_of(off // GROUP, GPC), GPC)], idx_v)
        copies = [
            pltpu.async_copy(
                tok_hbm.at[idx_v.at[k]],
                rows_v.at[pl.ds(k * GROUP, GROUP)],
                sem,
            )
            for k in range(GPC)
        ]
        for c in copies:
            c.wait()

        def pos_body(t, carry2):
            p0 = pos_v[t, pl.ds(0, 16)]
            p1 = pos_v[t, pl.ds(16, 16)]
            for rep in range(REPS):
                j = rep * MAX_LEN + t
                rows_v[j, pl.ds(0, 16)] += p0
                rows_v[j, pl.ds(16, 16)] += p1
            return carry2

        lax.fori_loop(0, MAX_LEN, pos_body, 0, unroll=2)
        pltpu.sync_copy(rows_v, out_hbm.at[pl.ds(off, CHUNK)])
        return carry

    lax.fori_loop(0, NCHUNK, chunk_body, 0)


def kernel(seq, token_table, pos_table):
    seq2d = seq.reshape(NROWS // GROUP, GROUP)
    out = _embed(seq2d, token_table, pos_table)
    return out.reshape(BATCH, MAX_LEN, DEPTH)


# speedup vs baseline: 1.4889x; 1.0453x over previous
import functools

import jax
import jax.numpy as jnp
from jax import lax
from jax.experimental import pallas as pl
from jax.experimental.pallas import tpu as pltpu
from jax.experimental.pallas import tpu_sc as plsc

NC = 2
NS = 16
NW = NC * NS

BATCH = 4096
MAX_LEN = 200
DEPTH = 32
NROWS = BATCH * MAX_LEN
ROWS_PER_W = NROWS // NW

GROUP = 100
CHUNK = 1600
GPC = CHUNK // GROUP
NCHUNK = ROWS_PER_W // CHUNK
REPS = CHUNK // MAX_LEN

_mesh = plsc.VectorSubcoreMesh(core_axis_name="c", subcore_axis_name="s")


@functools.partial(
    pl.kernel,
    out_type=jax.ShapeDtypeStruct((NROWS, DEPTH), jnp.float32),
    mesh=_mesh,
    compiler_params=pltpu.CompilerParams(use_tc_tiling_on_sc=False),
    scratch_types=[
        pltpu.VMEM((2, GPC, GROUP), jnp.int32),
        pltpu.VMEM((2, CHUNK, DEPTH), jnp.float32),
        pltpu.VMEM((MAX_LEN, DEPTH), jnp.float32),
        pltpu.SemaphoreType.DMA,
        pltpu.SemaphoreType.DMA,
    ],
)
def _embed(seq_hbm, tok_hbm, pos_hbm, out_hbm, idx_v, rows_v, pos_v, gsem, wsem):
    wid = lax.axis_index("s") * NC + lax.axis_index("c")
    base = wid * ROWS_PER_W

    pltpu.sync_copy(pos_hbm, pos_v)

    def row_off(ci):
        return pl.multiple_of(base + ci * CHUNK, CHUNK)

    def stage_ids(ci, b):
        g0 = pl.multiple_of(row_off(ci) // GROUP, GPC)
        pltpu.sync_copy(seq_hbm.at[pl.ds(g0, GPC)], idx_v.at[b])

    def launch_gathers(ci, b):
        for k in range(GPC):
            pltpu.async_copy(
                tok_hbm.at[idx_v.at[b].at[k]],
                rows_v.at[b].at[pl.ds(k * GROUP, GROUP)],
                gsem,
            )

    def wait_gathers(b):
        pltpu.make_async_copy(tok_hbm.at[pl.ds(0, CHUNK)], rows_v.at[b], gsem).wait()

    stage_ids(0, 0)
    launch_gathers(0, 0)
    stage_ids(1, 1)
    launch_gathers(1, 1)

    def pair_body(i, carry):
        for b in (0, 1):
            ci = 2 * i + b
            wait_gathers(b)

            def pos_body(t, carry2):
                p0 = pos_v[t, pl.ds(0, 16)]
                p1 = pos_v[t, pl.ds(16, 16)]
                for rep in range(REPS):
                    j = rep * MAX_LEN + t
                    rows_v[b, j, pl.ds(0, 16)] += p0
                    rows_v[b, j, pl.ds(16, 16)] += p1
                return carry2

            lax.fori_loop(0, MAX_LEN, pos_body, 0, unroll=2)

            wb = pltpu.async_copy(rows_v.at[b], out_hbm.at[pl.ds(row_off(ci), CHUNK)], wsem)

            @pl.when(ci + 2 < NCHUNK)
            def _stage():
                stage_ids(ci + 2, b)

            wb.wait()

            @pl.when(ci + 2 < NCHUNK)
            def _launch():
                launch_gathers(ci + 2, b)

        return carry

    lax.fori_loop(0, NCHUNK // 2, pair_body, 0)


def kernel(seq, token_table, pos_table):
    seq2d = seq.reshape(NROWS // GROUP, GROUP)
    out = _embed(seq2d, token_table, pos_table)
    return out.reshape(BATCH, MAX_LEN, DEPTH)
